# Initial kernel scaffold; baseline (speedup 1.0000x reference)
import jax
import jax.numpy as jnp
from jax.experimental import pallas as pl


def kernel(text, lengths, emb, W1, b1, W2, b2):
    raise NotImplementedError("write your pallas kernel here")


# speedup vs baseline: 15.3494x; 15.3494x over previous
import functools

import jax
import jax.numpy as jnp
from jax import lax
from jax.experimental import pallas as pl
from jax.experimental.pallas import tpu as pltpu
from jax.experimental.pallas import tpu_sc as plsc

VOCAB = 1000000
EMB = 32
HID = 128
OUT = 2
B = 16384
S = 200

NC = 2
NS = 16
NW = NC * NS
BPW = B // NW
G = 4
ROWS = G * S
NG = BPW // G


def _pooled_sums_sc(text_flat, emb):
    mesh = plsc.VectorSubcoreMesh(
        core_axis_name="c", subcore_axis_name="s", num_cores=NC, num_subcores=NS
    )

    @functools.partial(
        pl.kernel,
        out_type=jax.ShapeDtypeStruct((B, EMB), jnp.float32),
        mesh=mesh,
        compiler_params=pltpu.CompilerParams(use_tc_tiling_on_sc=False),
        scratch_types=[
            pltpu.VMEM((ROWS,), jnp.int32),
            pltpu.VMEM((ROWS,), jnp.int32),
            pltpu.VMEM((ROWS, EMB), jnp.float32),
            pltpu.VMEM((ROWS, EMB), jnp.float32),
            pltpu.VMEM((BPW, EMB), jnp.float32),
            pltpu.SemaphoreType.DMA,
            pltpu.SemaphoreType.DMA,
            pltpu.SemaphoreType.DMA,
            pltpu.SemaphoreType.DMA,
        ],
    )
    def k(text_hbm, emb_hbm, out_hbm, idx0, idx1, rows0, rows1, outb,
          isem0, isem1, rsem0, rsem1):
        wid = lax.axis_index("c") * NS + lax.axis_index("s")
        flat_base = wid * (BPW * S)

        idx_refs = (idx0, idx1)
        row_refs = (rows0, rows1)
        isems = (isem0, isem1)
        rsems = (rsem0, rsem1)

        def idx_start(g, buf):
            src = text_hbm.at[pl.ds(flat_base + g * ROWS, ROWS)]
            pltpu.async_copy(src, idx_refs[buf], isems[buf])

        def idx_wait(buf):
            pltpu.make_async_copy(
                text_hbm.at[pl.ds(0, ROWS)], idx_refs[buf], isems[buf]
            ).wait()

        def gather_start(buf):
            pltpu.async_copy(emb_hbm.at[idx_refs[buf]], row_refs[buf], rsems[buf])

        def gather_wait(buf):
            pltpu.make_async_copy(
                emb_hbm.at[idx_refs[buf]], row_refs[buf], rsems[buf]
            ).wait()

        def reduce(g, buf):
            rows = row_refs[buf]
            for i in range(G):
                def body(j, accs):
                    a0, a1 = accs
                    base = i * S + j * 8
                    for u in range(8):
                        a0 = a0 + rows[base + u, 0:16]
                        a1 = a1 + rows[base + u, 16:32]
                    return (a0, a1)
                a0, a1 = lax.fori_loop(
                    0, S // 8, body,
                    (jnp.zeros((16,), jnp.float32), jnp.zeros((16,), jnp.float32)),
                )
                outb[g * G + i, 0:16] = a0
                outb[g * G + i, 16:32] = a1

        idx_start(0, 0)
        idx_start(1, 1)
        idx_wait(0)
        gather_start(0)

        def step(p, carry):
            gather_wait(0)
            idx_start(2 * p + 2, 0)
            idx_wait(1)
            gather_start(1)
            reduce(2 * p, 0)
            gather_wait(1)
            idx_start(2 * p + 3, 1)
            idx_wait(0)
            gather_start(0)
            reduce(2 * p + 1, 1)
            return carry

        lax.fori_loop(0, NG // 2 - 1, step, 0)

        p = NG // 2 - 1
        gather_wait(0)
        idx_wait(1)
        gather_start(1)
        reduce(2 * p, 0)
        gather_wait(1)
        reduce(2 * p + 1, 1)

        pltpu.sync_copy(outb, out_hbm.at[pl.ds(wid * BPW, BPW)])

    return k(text_flat, emb)


def _mlp_block(x_ref, w1t_ref, b1_ref, w2t_ref, b2_ref, o_ref):
    x = x_ref[...] * (1.0 / S)
    h = jnp.dot(x, w1t_ref[...], preferred_element_type=jnp.float32) + b1_ref[...]
    h = jnp.maximum(h, 0.0)
    o_ref[...] = (
        jnp.dot(h, w2t_ref[...], preferred_element_type=jnp.float32) + b2_ref[...]
    )


def _mlp_tc(pooled_sums, W1, b1, W2, b2):
    blk = 2048
    grid = (B // blk,)
    return pl.pallas_call(
        _mlp_block,
        grid=grid,
        in_specs=[
            pl.BlockSpec((blk, EMB), lambda i: (i, 0)),
            pl.BlockSpec((EMB, HID), lambda i: (0, 0)),
            pl.BlockSpec((1, HID), lambda i: (0, 0)),
            pl.BlockSpec((HID, OUT), lambda i: (0, 0)),
            pl.BlockSpec((1, OUT), lambda i: (0, 0)),
        ],
        out_specs=pl.BlockSpec((blk, OUT), lambda i: (i, 0)),
        out_shape=jax.ShapeDtypeStruct((B, OUT), jnp.float32),
    )(pooled_sums, W1.T, b1.reshape(1, HID), W2.T, b2.reshape(1, OUT))


def kernel(text, lengths, emb, W1, b1, W2, b2):
    del lengths
    pooled_sums = _pooled_sums_sc(text.reshape(-1).astype(jnp.int32), emb)
    return _mlp_tc(pooled_sums, W1, b1, W2, b2)


# speedup vs baseline: 27.2356x; 1.7744x over previous
import functools

import jax
import jax.numpy as jnp
from jax import lax
from jax.experimental import pallas as pl
from jax.experimental.pallas import tpu as pltpu
from jax.experimental.pallas import tpu_sc as plsc

VOCAB = 1000000
EMB = 32
HID = 128
OUT = 2
B = 16384
S = 200

NC = 2
NS = 16
NW = NC * NS
BPW = B // NW
G = 4
ROWS = G * S
NG = BPW // G


def _pooled_sums_sc(text_flat, emb):
    mesh = plsc.VectorSubcoreMesh(
        core_axis_name="c", subcore_axis_name="s", num_cores=NC, num_subcores=NS
    )

    @functools.partial(
        pl.kernel,
        out_type=jax.ShapeDtypeStruct((B, EMB), jnp.float32),
        mesh=mesh,
        compiler_params=pltpu.CompilerParams(use_tc_tiling_on_sc=False),
        scratch_types=[
            pltpu.VMEM((ROWS,), jnp.int32),
            pltpu.VMEM((ROWS,), jnp.int32),
            pltpu.VMEM((ROWS, EMB), jnp.float32),
            pltpu.VMEM((ROWS, EMB), jnp.float32),
            pltpu.VMEM((BPW, EMB), jnp.float32),
            pltpu.SemaphoreType.DMA,
            pltpu.SemaphoreType.DMA,
            pltpu.SemaphoreType.DMA,
            pltpu.SemaphoreType.DMA,
        ],
    )
    def k(text_hbm, emb_hbm, out_hbm, idx0, idx1, rows0, rows1, outb,
          isem0, isem1, rsem0, rsem1):
        wid = lax.axis_index("c") * NS + lax.axis_index("s")
        flat_base = wid * (BPW * S)

        idx_refs = (idx0, idx1)
        row_refs = (rows0, rows1)
        isems = (isem0, isem1)
        rsems = (rsem0, rsem1)

        def idx_start(g, buf):
            src = text_hbm.at[pl.ds(flat_base + g * ROWS, ROWS)]
            pltpu.async_copy(src, idx_refs[buf], isems[buf])

        def idx_wait(buf):
            pltpu.make_async_copy(
                text_hbm.at[pl.ds(0, ROWS)], idx_refs[buf], isems[buf]
            ).wait()

        def gather_start(buf):
            pltpu.async_copy(emb_hbm.at[idx_refs[buf]], row_refs[buf], rsems[buf])

        def gather_wait(buf):
            pltpu.make_async_copy(
                emb_hbm.at[idx_refs[buf]], row_refs[buf], rsems[buf]
            ).wait()

        def reduce(g, buf):
            rows = row_refs[buf]
            for i in range(G):
                def body(j, accs):
                    a0, a1 = accs
                    base = i * S + j * 8
                    for u in range(8):
                        a0 = a0 + rows[base + u, 0:16]
                        a1 = a1 + rows[base + u, 16:32]
                    return (a0, a1)
                a0, a1 = lax.fori_loop(
                    0, S // 8, body,
                    (jnp.zeros((16,), jnp.float32), jnp.zeros((16,), jnp.float32)),
                )
                outb[g * G + i, 0:16] = a0
                outb[g * G + i, 16:32] = a1

        idx_start(0, 0)
        idx_start(1, 1)
        idx_wait(0)
        gather_start(0)

        def step(p, carry):
            gather_wait(0)
            idx_start(2 * p + 2, 0)
            idx_wait(1)
            gather_start(1)
            reduce(2 * p, 0)
            gather_wait(1)
            idx_start(2 * p + 3, 1)
            idx_wait(0)
            gather_start(0)
            reduce(2 * p + 1, 1)
            return carry

        lax.fori_loop(0, NG // 2 - 1, step, 0)

        p = NG // 2 - 1
        gather_wait(0)
        idx_wait(1)
        gather_start(1)
        reduce(2 * p, 0)
        gather_wait(1)
        reduce(2 * p + 1, 1)

        pltpu.sync_copy(outb, out_hbm.at[pl.ds(wid * BPW, BPW)])

    return k(text_flat, emb)


TRC = 8192
TRQ = TRC // 4


def _tr_block(x_ref, o_ref):
    x = x_ref[...]
    z = jnp.concatenate([x[:, k * TRQ:(k + 1) * TRQ] for k in range(4)], axis=0)
    o_ref[...] = jnp.transpose(z)


def _emb_linear_tc(embT):
    nblk = (VOCAB + TRC - 1) // TRC
    return pl.pallas_call(
        _tr_block,
        grid=(nblk,),
        in_specs=[pl.BlockSpec((EMB, TRC), lambda i: (0, i))],
        out_specs=pl.BlockSpec((TRC // 4, 128), lambda i: (i, 0)),
        out_shape=jax.ShapeDtypeStruct((nblk * TRC // 4, 128), jnp.float32),
    )(embT)


def _mlp_block(x_ref, w1t_ref, b1_ref, w2t_ref, b2_ref, o_ref):
    x = x_ref[...] * (1.0 / S)
    h = jnp.dot(x, w1t_ref[...], preferred_element_type=jnp.float32) + b1_ref[...]
    h = jnp.maximum(h, 0.0)
    o_ref[...] = (
        jnp.dot(h, w2t_ref[...], preferred_element_type=jnp.float32) + b2_ref[...]
    )


def _mlp_tc(pooled_sums, W1, b1, W2, b2):
    blk = 2048
    grid = (B // blk,)
    return pl.pallas_call(
        _mlp_block,
        grid=grid,
        in_specs=[
            pl.BlockSpec((blk, EMB), lambda i: (i, 0)),
            pl.BlockSpec((EMB, HID), lambda i: (0, 0)),
            pl.BlockSpec((1, HID), lambda i: (0, 0)),
            pl.BlockSpec((HID, OUT), lambda i: (0, 0)),
            pl.BlockSpec((1, OUT), lambda i: (0, 0)),
        ],
        out_specs=pl.BlockSpec((blk, OUT), lambda i: (i, 0)),
        out_shape=jax.ShapeDtypeStruct((B, OUT), jnp.float32),
    )(pooled_sums, W1.T, b1.reshape(1, HID), W2.T, b2.reshape(1, OUT))


def kernel(text, lengths, emb, W1, b1, W2, b2):
    del lengths
    table = _emb_linear_tc(emb.T)
    nrec = table.shape[0] * (128 // EMB)
    table = table.reshape(-1).reshape(nrec, EMB)
    tf = text.reshape(-1).astype(jnp.int32)
    rec = (tf & jnp.int32(-TRC)) | ((tf & jnp.int32(TRQ - 1)) << 2) | (
        (tf >> 11) & jnp.int32(3)
    )
    pooled_sums = _pooled_sums_sc(rec, table)
    return _mlp_tc(pooled_sums, W1, b1, W2, b2)


# speedup vs baseline: 29.4009x; 1.0795x over previous
import functools

import jax
import jax.numpy as jnp
from jax import lax
from jax.experimental import pallas as pl
from jax.experimental.pallas import tpu as pltpu
from jax.experimental.pallas import tpu_sc as plsc

VOCAB = 1000000
EMB = 32
HID = 128
OUT = 2
B = 16384
S = 200

NC = 2
NS = 16
NW = NC * NS
BPW = B // NW
G = 4
ROWS = G * S
NG = BPW // G

TRC = 8192
TRQ = TRC // 8
NBLK = (VOCAB + TRC - 1) // TRC
NREC = NBLK * TRC


def _tr_block(x_ref, o_ref):
    x = x_ref[...]
    packs = []
    for q in range(8):
        s = x[:, q * TRQ:(q + 1) * TRQ].astype(jnp.bfloat16)
        lo = jax.lax.bitcast_convert_type(s[0:16], jnp.uint16).astype(jnp.uint32)
        hi = jax.lax.bitcast_convert_type(s[16:32], jnp.uint16).astype(jnp.uint32)
        packs.append((lo | (hi << 16)).astype(jnp.int32))
    z = jnp.concatenate(packs, axis=0)
    o_ref[...] = jnp.transpose(z)


def _pack_table_tc(embT):
    return pl.pallas_call(
        _tr_block,
        grid=(NBLK,),
        in_specs=[pl.BlockSpec((EMB, TRC), lambda i: (0, i))],
        out_specs=pl.BlockSpec((TRQ, 128), lambda i: (i, 0)),
        out_shape=jax.ShapeDtypeStruct((NBLK * TRQ, 128), jnp.int32),
    )(embT)


def _pooled_sums_sc(rec_flat, table):
    mesh = plsc.VectorSubcoreMesh(
        core_axis_name="c", subcore_axis_name="s", num_cores=NC, num_subcores=NS
    )

    @functools.partial(
        pl.kernel,
        out_type=jax.ShapeDtypeStruct((B, EMB), jnp.float32),
        mesh=mesh,
        compiler_params=pltpu.CompilerParams(
            use_tc_tiling_on_sc=False, needs_layout_passes=False
        ),
        scratch_types=[
            pltpu.VMEM((ROWS,), jnp.int32),
            pltpu.VMEM((ROWS,), jnp.int32),
            pltpu.VMEM((ROWS, 16), jnp.int32),
            pltpu.VMEM((ROWS, 16), jnp.int32),
            pltpu.VMEM((BPW, EMB), jnp.float32),
            pltpu.SemaphoreType.DMA,
            pltpu.SemaphoreType.DMA,
            pltpu.SemaphoreType.DMA,
            pltpu.SemaphoreType.DMA,
        ],
    )
    def k(rec_hbm, tab_hbm, out_hbm, idx0, idx1, rows0, rows1, outb,
          isem0, isem1, rsem0, rsem1):
        wid = lax.axis_index("c") * NS + lax.axis_index("s")
        flat_base = wid * (BPW * S)

        idx_refs = (idx0, idx1)
        row_refs = (rows0, rows1)
        isems = (isem0, isem1)
        rsems = (rsem0, rsem1)

        def idx_start(g, buf):
            src = rec_hbm.at[pl.ds(flat_base + g * ROWS, ROWS)]
            pltpu.async_copy(src, idx_refs[buf], isems[buf])

        def idx_wait(buf):
            pltpu.make_async_copy(
                rec_hbm.at[pl.ds(0, ROWS)], idx_refs[buf], isems[buf]
            ).wait()

        def gather_start(buf):
            pltpu.async_copy(tab_hbm.at[idx_refs[buf]], row_refs[buf], rsems[buf])

        def gather_wait(buf):
            pltpu.make_async_copy(
                tab_hbm.at[idx_refs[buf]], row_refs[buf], rsems[buf]
            ).wait()

        def reduce(g, buf):
            rows = row_refs[buf]
            for i in range(G):
                def body(j, accs):
                    a0, a1 = accs
                    base = i * S + j * 8
                    for u in range(8):
                        v = plsc.bitcast(rows[base + u, 0:16], jnp.bfloat16)
                        lo, hi = plsc.unpack(v, format=plsc.PackFormat.INTERLEAVED)
                        a0 = a0 + lo
                        a1 = a1 + hi
                    return (a0, a1)
                a0, a1 = lax.fori_loop(
                    0, S // 8, body,
                    (jnp.zeros((16,), jnp.float32), jnp.zeros((16,), jnp.float32)),
                )
                outb[g * G + i, 0:16] = a0
                outb[g * G + i, 16:32] = a1

        idx_start(0, 0)
        idx_start(1, 1)
        idx_wait(0)
        gather_start(0)

        def step(p, carry):
            gather_wait(0)
            idx_start(2 * p + 2, 0)
            idx_wait(1)
            gather_start(1)
            reduce(2 * p, 0)
            gather_wait(1)
            idx_start(2 * p + 3, 1)
            idx_wait(0)
            gather_start(0)
            reduce(2 * p + 1, 1)
            return carry

        lax.fori_loop(0, NG // 2 - 1, step, 0)

        p = NG // 2 - 1
        gather_wait(0)
        idx_wait(1)
        gather_start(1)
        reduce(2 * p, 0)
        gather_wait(1)
        reduce(2 * p + 1, 1)

        pltpu.sync_copy(outb, out_hbm.at[pl.ds(wid * BPW, BPW)])

    return k(rec_flat, table)


def _mlp_block(x_ref, w1t_ref, b1_ref, w2t_ref, b2_ref, o_ref):
    x = x_ref[...] * (1.0 / S)
    h = jnp.dot(x, w1t_ref[...], preferred_element_type=jnp.float32) + b1_ref[...]
    h = jnp.maximum(h, 0.0)
    o_ref[...] = (
        jnp.dot(h, w2t_ref[...], preferred_element_type=jnp.float32) + b2_ref[...]
    )


def _mlp_tc(pooled_sums, W1, b1, W2, b2):
    blk = 2048
    grid = (B // blk,)
    return pl.pallas_call(
        _mlp_block,
        grid=grid,
        in_specs=[
            pl.BlockSpec((blk, EMB), lambda i: (i, 0)),
            pl.BlockSpec((EMB, HID), lambda i: (0, 0)),
            pl.BlockSpec((1, HID), lambda i: (0, 0)),
            pl.BlockSpec((HID, OUT), lambda i: (0, 0)),
            pl.BlockSpec((1, OUT), lambda i: (0, 0)),
        ],
        out_specs=pl.BlockSpec((blk, OUT), lambda i: (i, 0)),
        out_shape=jax.ShapeDtypeStruct((B, OUT), jnp.float32),
    )(pooled_sums, W1.T, b1.reshape(1, HID), W2.T, b2.reshape(1, OUT))


def kernel(text, lengths, emb, W1, b1, W2, b2):
    del lengths
    packed = _pack_table_tc(emb.T)
    table = packed.reshape(-1).reshape(NREC, 16)
    tf = text.reshape(-1).astype(jnp.int32)
    rec = (tf & jnp.int32(-TRC)) | ((tf & jnp.int32(TRQ - 1)) << 3) | (
        (tf >> 10) & jnp.int32(7)
    )
    pooled_sums = _pooled_sums_sc(rec, table)
    return _mlp_tc(pooled_sums, W1, b1, W2, b2)


# speedup vs baseline: 31.9251x; 1.0859x over previous
import functools

import jax
import jax.numpy as jnp
from jax import lax
from jax.experimental import pallas as pl
from jax.experimental.pallas import tpu as pltpu
from jax.experimental.pallas import tpu_sc as plsc

VOCAB = 1000000
EMB = 32
HID = 128
OUT = 2
B = 16384
S = 200

NC = 2
NS = 16
NW = NC * NS
BPW = B // NW
G = 4
ROWS = G * S
NG = BPW // G

TRC = 16384
TRQ = TRC // 8
NBLK = (VOCAB + TRC - 1) // TRC
NREC = NBLK * TRC


def _tr_block(x_ref, o_ref):
    x = x_ref[...]
    packs = []
    for q in range(8):
        s = x[:, q * TRQ:(q + 1) * TRQ].astype(jnp.bfloat16)
        lo = jax.lax.bitcast_convert_type(s[0:16], jnp.uint16).astype(jnp.uint32)
        hi = jax.lax.bitcast_convert_type(s[16:32], jnp.uint16).astype(jnp.uint32)
        packs.append((lo | (hi << 16)).astype(jnp.int32))
    z = jnp.concatenate(packs, axis=0)
    o_ref[...] = jnp.transpose(z)


def _pack_table_tc(embT):
    return pl.pallas_call(
        _tr_block,
        grid=(NBLK,),
        in_specs=[pl.BlockSpec((EMB, TRC), lambda i: (0, i))],
        out_specs=pl.BlockSpec((TRQ, 128), lambda i: (i, 0)),
        out_shape=jax.ShapeDtypeStruct((NBLK * TRQ, 128), jnp.int32),
    )(embT)


def _pooled_sums_sc(rec_flat, table):
    mesh = plsc.VectorSubcoreMesh(
        core_axis_name="c", subcore_axis_name="s", num_cores=NC, num_subcores=NS
    )

    @functools.partial(
        pl.kernel,
        out_type=jax.ShapeDtypeStruct((B, EMB), jnp.float32),
        mesh=mesh,
        compiler_params=pltpu.CompilerParams(
            use_tc_tiling_on_sc=False, needs_layout_passes=False
        ),
        scratch_types=[
            pltpu.VMEM((ROWS,), jnp.int32),
            pltpu.VMEM((ROWS,), jnp.int32),
            pltpu.VMEM((ROWS, 16), jnp.int32),
            pltpu.VMEM((ROWS, 16), jnp.int32),
            pltpu.VMEM((BPW, EMB), jnp.float32),
            pltpu.SemaphoreType.DMA,
            pltpu.SemaphoreType.DMA,
            pltpu.SemaphoreType.DMA,
            pltpu.SemaphoreType.DMA,
        ],
    )
    def k(rec_hbm, tab_hbm, out_hbm, idx0, idx1, rows0, rows1, outb,
          isem0, isem1, rsem0, rsem1):
        wid = lax.axis_index("c") * NS + lax.axis_index("s")
        flat_base = wid * (BPW * S)

        idx_refs = (idx0, idx1)
        row_refs = (rows0, rows1)
        isems = (isem0, isem1)
        rsems = (rsem0, rsem1)

        def idx_start(g, buf):
            src = rec_hbm.at[pl.ds(flat_base + g * ROWS, ROWS)]
            pltpu.async_copy(src, idx_refs[buf], isems[buf])

        def idx_wait(buf):
            pltpu.make_async_copy(
                rec_hbm.at[pl.ds(0, ROWS)], idx_refs[buf], isems[buf]
            ).wait()

        def gather_start(buf):
            pltpu.async_copy(tab_hbm.at[idx_refs[buf]], row_refs[buf], rsems[buf])

        def gather_wait(buf):
            pltpu.make_async_copy(
                tab_hbm.at[idx_refs[buf]], row_refs[buf], rsems[buf]
            ).wait()

        def reduce(g, buf):
            rows = row_refs[buf]
            for i in range(G):
                def body(j, accs):
                    a0, a1 = accs
                    base = i * S + j * 8
                    v = [
                        plsc.bitcast(rows[base + u, 0:16], jnp.bfloat16)
                        for u in range(8)
                    ]
                    w = [v[2 * u] + v[2 * u + 1] for u in range(4)]
                    x2 = [w[0] + w[1], w[2] + w[3]]
                    t = x2[0] + x2[1]
                    lo, hi = plsc.unpack(t, format=plsc.PackFormat.INTERLEAVED)
                    return (a0 + lo, a1 + hi)
                a0, a1 = lax.fori_loop(
                    0, S // 8, body,
                    (jnp.zeros((16,), jnp.float32), jnp.zeros((16,), jnp.float32)),
                )
                outb[g * G + i, 0:16] = a0
                outb[g * G + i, 16:32] = a1

        idx_start(0, 0)
        idx_start(1, 1)
        idx_wait(0)
        gather_start(0)

        def step(p, carry):
            gather_wait(0)
            idx_start(2 * p + 2, 0)
            idx_wait(1)
            gather_start(1)
            reduce(2 * p, 0)
            gather_wait(1)
            idx_start(2 * p + 3, 1)
            idx_wait(0)
            gather_start(0)
            reduce(2 * p + 1, 1)
            return carry

        lax.fori_loop(0, NG // 2 - 1, step, 0)

        p = NG // 2 - 1
        gather_wait(0)
        idx_wait(1)
        gather_start(1)
        reduce(2 * p, 0)
        gather_wait(1)
        reduce(2 * p + 1, 1)

        pltpu.sync_copy(outb, out_hbm.at[pl.ds(wid * BPW, BPW)])

    return k(rec_flat, table)


def _mlp_block(x_ref, w1t_ref, b1_ref, w2t_ref, b2_ref, o_ref):
    x = x_ref[...] * (1.0 / S)
    h = jnp.dot(x, w1t_ref[...], preferred_element_type=jnp.float32) + b1_ref[...]
    h = jnp.maximum(h, 0.0)
    o_ref[...] = (
        jnp.dot(h, w2t_ref[...], preferred_element_type=jnp.float32) + b2_ref[...]
    )


def _mlp_tc(pooled_sums, W1, b1, W2, b2):
    blk = 2048
    grid = (B // blk,)
    return pl.pallas_call(
        _mlp_block,
        grid=grid,
        in_specs=[
            pl.BlockSpec((blk, EMB), lambda i: (i, 0)),
            pl.BlockSpec((EMB, HID), lambda i: (0, 0)),
            pl.BlockSpec((1, HID), lambda i: (0, 0)),
            pl.BlockSpec((HID, OUT), lambda i: (0, 0)),
            pl.BlockSpec((1, OUT), lambda i: (0, 0)),
        ],
        out_specs=pl.BlockSpec((blk, OUT), lambda i: (i, 0)),
        out_shape=jax.ShapeDtypeStruct((B, OUT), jnp.float32),
    )(pooled_sums, W1.T, b1.reshape(1, HID), W2.T, b2.reshape(1, OUT))


def kernel(text, lengths, emb, W1, b1, W2, b2):
    del lengths
    packed = _pack_table_tc(emb.T)
    table = packed.reshape(-1).reshape(NREC, 16)
    tf = text.reshape(-1).astype(jnp.int32)
    rec = (tf & jnp.int32(-TRC)) | ((tf & jnp.int32(TRQ - 1)) << 3) | (
        (tf >> (TRQ.bit_length() - 1)) & jnp.int32(7)
    )
    pooled_sums = _pooled_sums_sc(rec, table)
    return _mlp_tc(pooled_sums, W1, b1, W2, b2)


# speedup vs baseline: 41.0928x; 1.2872x over previous
import functools

import jax
import jax.numpy as jnp
from jax import lax
from jax.experimental import pallas as pl
from jax.experimental.pallas import tpu as pltpu
from jax.experimental.pallas import tpu_sc as plsc

VOCAB = 1000000
EMB = 32
HID = 128
OUT = 2
B = 16384
S = 200

NC = 2
NS = 16
NW = NC * NS
BPW = B // NW
G = 4
ROWS = G * S
NG = BPW // G

TRC = 16384
TRQ = TRC // 8
NBLK = (VOCAB + TRC - 1) // TRC
NREC = NBLK * TRC


def _tr_block(x_ref, o_ref):
    x = x_ref[...]
    packs = []
    for q in range(8):
        s = x[:, q * TRQ:(q + 1) * TRQ].astype(jnp.bfloat16)
        lo = jax.lax.bitcast_convert_type(s[0:16], jnp.uint16).astype(jnp.uint32)
        hi = jax.lax.bitcast_convert_type(s[16:32], jnp.uint16).astype(jnp.uint32)
        packs.append((lo | (hi << 16)).astype(jnp.int32))
    z = jnp.concatenate(packs, axis=0)
    o_ref[...] = jnp.transpose(z)


def _pack_table_tc(embT):
    return pl.pallas_call(
        _tr_block,
        grid=(NBLK,),
        in_specs=[pl.BlockSpec((EMB, TRC), lambda i: (0, i))],
        out_specs=pl.BlockSpec((TRQ, 128), lambda i: (i, 0)),
        out_shape=jax.ShapeDtypeStruct((NBLK * TRQ, 128), jnp.int32),
    )(embT)


def _pooled_sums_sc(rec_flat, table):
    mesh = plsc.VectorSubcoreMesh(
        core_axis_name="c", subcore_axis_name="s", num_cores=NC, num_subcores=NS
    )

    @functools.partial(
        pl.kernel,
        out_type=jax.ShapeDtypeStruct((B, EMB), jnp.float32),
        mesh=mesh,
        compiler_params=pltpu.CompilerParams(
            use_tc_tiling_on_sc=False, needs_layout_passes=False
        ),
        scratch_types=[
            pltpu.VMEM((ROWS,), jnp.int32),
            pltpu.VMEM((ROWS,), jnp.int32),
            pltpu.VMEM((ROWS,), jnp.int32),
            pltpu.VMEM((ROWS,), jnp.int32),
            pltpu.VMEM((ROWS, 16), jnp.int32),
            pltpu.VMEM((ROWS, 16), jnp.int32),
            pltpu.VMEM((ROWS, 16), jnp.int32),
            pltpu.VMEM((ROWS, 16), jnp.int32),
            pltpu.VMEM((BPW, EMB), jnp.float32),
            pltpu.SemaphoreType.DMA,
            pltpu.SemaphoreType.DMA,
            pltpu.SemaphoreType.DMA,
            pltpu.SemaphoreType.DMA,
            pltpu.SemaphoreType.DMA,
            pltpu.SemaphoreType.DMA,
            pltpu.SemaphoreType.DMA,
            pltpu.SemaphoreType.DMA,
        ],
    )
    def k(rec_hbm, tab_hbm, out_hbm, idx0, idx1, idx2, idx3,
          rows0, rows1, rows2, rows3, outb,
          isem0, isem1, isem2, isem3, rsem0, rsem1, rsem2, rsem3):
        wid = lax.axis_index("c") * NS + lax.axis_index("s")
        flat_base = wid * (BPW * S)

        idx_refs = (idx0, idx1, idx2, idx3)
        row_refs = (rows0, rows1, rows2, rows3)
        isems = (isem0, isem1, isem2, isem3)
        rsems = (rsem0, rsem1, rsem2, rsem3)

        def idx_start(g, buf):
            src = rec_hbm.at[pl.ds(flat_base + g * ROWS, ROWS)]
            pltpu.async_copy(src, idx_refs[buf], isems[buf])

        def idx_wait(buf):
            pltpu.make_async_copy(
                rec_hbm.at[pl.ds(0, ROWS)], idx_refs[buf], isems[buf]
            ).wait()

        def gather_start(buf):
            pltpu.async_copy(tab_hbm.at[idx_refs[buf]], row_refs[buf], rsems[buf])

        def gather_wait(buf):
            pltpu.make_async_copy(
                tab_hbm.at[idx_refs[buf]], row_refs[buf], rsems[buf]
            ).wait()

        def reduce(g, buf):
            rows = row_refs[buf]
            for i in range(G):
                def body(j, accs):
                    a0, a1 = accs
                    base = i * S + j * 8
                    v = [
                        plsc.bitcast(rows[base + u, 0:16], jnp.bfloat16)
                        for u in range(8)
                    ]
                    w = [v[2 * u] + v[2 * u + 1] for u in range(4)]
                    x2 = [w[0] + w[1], w[2] + w[3]]
                    t = x2[0] + x2[1]
                    lo, hi = plsc.unpack(t, format=plsc.PackFormat.INTERLEAVED)
                    return (a0 + lo, a1 + hi)
                a0, a1 = lax.fori_loop(
                    0, S // 8, body,
                    (jnp.zeros((16,), jnp.float32), jnp.zeros((16,), jnp.float32)),
                )
                outb[g * G + i, 0:16] = a0
                outb[g * G + i, 16:32] = a1

        for b in range(4):
            idx_start(b, b)
        for b in range(3):
            idx_wait(b)
            gather_start(b)

        def phase(g, b, issue_idx, issue_gather):
            gather_wait(b)
            if issue_idx:
                idx_start(g + 4, b)
            if issue_gather:
                idx_wait((b + 3) % 4)
                gather_start((b + 3) % 4)
            reduce(g, b)

        def step(p, carry):
            for kk in range(4):
                phase(4 * p + kk, kk, True, True)
            return carry

        lax.fori_loop(0, NG // 4 - 1, step, 0)

        g0 = NG - 4
        phase(g0 + 0, 0, False, True)
        phase(g0 + 1, 1, False, False)
        phase(g0 + 2, 2, False, False)
        phase(g0 + 3, 3, False, False)

        pltpu.sync_copy(outb, out_hbm.at[pl.ds(wid * BPW, BPW)])

    return k(rec_flat, table)


def _mlp_block(x_ref, w1t_ref, b1_ref, w2t_ref, b2_ref, o_ref):
    x = x_ref[...] * (1.0 / S)
    h = jnp.dot(x, w1t_ref[...], preferred_element_type=jnp.float32) + b1_ref[...]
    h = jnp.maximum(h, 0.0)
    o_ref[...] = (
        jnp.dot(h, w2t_ref[...], preferred_element_type=jnp.float32) + b2_ref[...]
    )


def _mlp_tc(pooled_sums, W1, b1, W2, b2):
    blk = 2048
    grid = (B // blk,)
    return pl.pallas_call(
        _mlp_block,
        grid=grid,
        in_specs=[
            pl.BlockSpec((blk, EMB), lambda i: (i, 0)),
            pl.BlockSpec((EMB, HID), lambda i: (0, 0)),
            pl.BlockSpec((1, HID), lambda i: (0, 0)),
            pl.BlockSpec((HID, OUT), lambda i: (0, 0)),
            pl.BlockSpec((1, OUT), lambda i: (0, 0)),
        ],
        out_specs=pl.BlockSpec((blk, OUT), lambda i: (i, 0)),
        out_shape=jax.ShapeDtypeStruct((B, OUT), jnp.float32),
    )(pooled_sums, W1.T, b1.reshape(1, HID), W2.T, b2.reshape(1, OUT))


def kernel(text, lengths, emb, W1, b1, W2, b2):
    del lengths
    packed = _pack_table_tc(emb.T)
    table = packed.reshape(-1).reshape(NREC, 16)
    tf = text.reshape(-1).astype(jnp.int32)
    rec = (tf & jnp.int32(-TRC)) | ((tf & jnp.int32(TRQ - 1)) << 3) | (
        (tf >> (TRQ.bit_length() - 1)) & jnp.int32(7)
    )
    pooled_sums = _pooled_sums_sc(rec, table)
    return _mlp_tc(pooled_sums, W1, b1, W2, b2)
